# baseline (speedup 1.0000x reference)
import dataclasses

import jax
import jax.numpy as jnp
from jax.experimental import pallas as pl
from jax.experimental.pallas import tpu as pltpu
from jax.experimental.pallas import tpu_sc as plsc

_WINDOW = 256
_NBUF = 2
_WORKERS = 32


def kernel(input_ids, table):
    batch, hist = input_ids.shape
    vocab, dim = table.shape
    num_indices = batch * hist
    per_worker = num_indices // _WORKERS
    steps = per_worker // _WINDOW
    assert per_worker % _WINDOW == 0 and steps % _NBUF == 0

    mesh = plsc.VectorSubcoreMesh(core_axis_name="c", subcore_axis_name="s")
    params = pltpu.CompilerParams(use_tc_tiling_on_sc=False)

    @jax.jit
    def run(table, idx):
        @pl.kernel(
            out_type=jax.ShapeDtypeStruct((num_indices, dim), table.dtype),
            mesh=mesh,
            compiler_params=params,
            scratch_types=[
                pltpu.VMEM((per_worker,), jnp.int32),
                pltpu.VMEM((_NBUF, _WINDOW, dim), table.dtype),
                pltpu.SemaphoreType.DMA((_NBUF,)),
                pltpu.SemaphoreType.DMA((_NBUF,)),
            ],
        )
        def gather_kernel(table_hbm, idx_hbm, out_hbm, idx_v, rows_v, gsem, wsem):
            wid = jax.lax.axis_index("s") * 2 + jax.lax.axis_index("c")
            wbase = wid * per_worker
            pltpu.sync_copy(idx_hbm.at[pl.ds(wbase, per_worker)], idx_v)

            @pl.loop(0, steps, step=_NBUF)
            def _(g):
                gathers = []
                for b in range(_NBUF):
                    base = (g + b) * _WINDOW
                    idx_win = idx_v.at[pl.ds(base, _WINDOW)]
                    gathers.append(
                        pltpu.async_copy(
                            table_hbm.at[idx_win], rows_v.at[b], gsem.at[b]
                        )
                    )
                writes = []
                for b in range(_NBUF):
                    base = (g + b) * _WINDOW
                    gathers[b].wait()
                    writes.append(
                        pltpu.async_copy(
                            rows_v.at[b],
                            out_hbm.at[pl.ds(wbase + base, _WINDOW)],
                            wsem.at[b],
                        )
                    )
                for w in writes:
                    w.wait()

        out = gather_kernel(table, idx)
        return out.reshape(batch, hist, dim)

    return run(table, input_ids.reshape(num_indices))
